# baseline (speedup 1.0000x reference)
import jax
import jax.numpy as jnp
from jax.experimental import pallas as pl
from jax.experimental.pallas import tpu as pltpu

B = 8
DIM = 384
HEADS = 6
HD = 64
SCALE = HD ** -0.5
NEG = -1e30


def _ln(x, s, b):
    m = jnp.mean(x, axis=-1, keepdims=True)
    d = x - m
    v = jnp.mean(d * d, axis=-1, keepdims=True)
    return d / jnp.sqrt(v + 1e-5) * s + b


def _gelu(x):
    return 0.5 * x * (jax.lax.erf(x / 1.4142135623730951) + 1.0)


_DN = (((1,), (0,)), ((), ()))
_DNT = (((1,), (1,)), ((), ()))


def _split(a):
    ah = a.astype(jnp.bfloat16)
    al = (a - ah.astype(jnp.float32)).astype(jnp.bfloat16)
    return ah, al


def _dot(a, b):
    ah, al = _split(a)
    bh, bl = _split(b)

    def d(x, y):
        return jax.lax.dot_general(x, y, _DN,
                                   preferred_element_type=jnp.float32)

    return d(ah, bh) + (d(ah, bl) + d(al, bh))


def _dot_t(a, b):
    ah, al = _split(a)
    bh, bl = _split(b)

    def d(x, y):
        return jax.lax.dot_general(x, y, _DNT,
                                   preferred_element_type=jnp.float32)

    return d(ah, bh) + (d(ah, bl) + d(al, bh))


def _dot_hi(a, b):
    ah = a.astype(jnp.bfloat16)
    al = (a - ah.astype(jnp.float32)).astype(jnp.bfloat16)
    bh = b.astype(jnp.bfloat16)
    bl = (b - bh.astype(jnp.float32)).astype(jnp.bfloat16)

    def d(x, y):
        return jax.lax.dot_general(x, y, _DN,
                                   preferred_element_type=jnp.float32)

    return d(ah, bh) + (d(ah, bl) + d(al, bh))


def _attn_and_proj(hs, qkvs, cls_scr, n1s, n1b, qw, qb, pw, pb,
                   NP, N, M, CH):
    for c0 in range(0, M, CH):
        x = hs[c0:c0 + CH, :]
        qkvs[c0:c0 + CH, :] = _dot(_ln(x, n1s, n1b), qw) + qb
    cls_scr[...] = jnp.zeros((B, NP), jnp.float32)
    if NP > N:
        kmask = jax.lax.broadcasted_iota(jnp.int32, (NP, NP), 1) >= N
    else:
        kmask = None
    for b in range(B):
        r0 = b * NP
        for h in range(HEADS):
            q = qkvs[r0:r0 + NP, h * HD:(h + 1) * HD]
            k = qkvs[r0:r0 + NP, DIM + h * HD:DIM + (h + 1) * HD]
            v = qkvs[r0:r0 + NP, 2 * DIM + h * HD:2 * DIM + (h + 1) * HD]
            s = _dot_t(q, k) * SCALE
            if kmask is not None:
                s = jnp.where(kmask, NEG, s)
            s = s - jnp.max(s, axis=1, keepdims=True)
            e = jnp.exp(s)
            p = e / jnp.sum(e, axis=1, keepdims=True)
            cls_scr[b:b + 1, :] += p[0:1, :]
            qkvs[r0:r0 + NP, h * HD:(h + 1) * HD] = _dot(p, v)
    for c0 in range(0, M, CH):
        hs[c0:c0 + CH, :] = (hs[c0:c0 + CH, :] +
                             _dot(qkvs[c0:c0 + CH, 0:DIM], pw) + pb)


def _mlp(hs, n2s, n2b, f1w, f1b, f2w, f2b, M, CH):
    for c0 in range(0, M, CH):
        x = hs[c0:c0 + CH, :]
        z = _ln(x, n2s, n2b)
        z = _dot(_gelu(_dot(z, f1w) + f1b), f2w) + f2b
        hs[c0:c0 + CH, :] = x + z


def _prune(hs, hs2, cls_scr, NP, N, LEFT, NNP):
    ij1 = jax.lax.broadcasted_iota(jnp.int32, (1, NP), 1)
    colv = (ij1 >= 1) & (ij1 <= N - 1)
    ik = jax.lax.broadcasted_iota(jnp.int32, (NP, NP), 0)
    ij = jax.lax.broadcasted_iota(jnp.int32, (NP, NP), 1)
    I2 = jax.lax.broadcasted_iota(jnp.int32, (NNP, NP), 0)
    J2 = jax.lax.broadcasted_iota(jnp.int32, (NNP, NP), 1)
    for b in range(B):
        a = cls_scr[b:b + 1, :] * (1.0 / HEADS)
        av = jnp.where(colv, a, NEG)
        Mb = jnp.broadcast_to(av, (NP, NP))
        MT = Mb.T
        cnt = (MT > Mb) | ((MT == Mb) & (ik < ij))
        rank = jnp.sum(cnt.astype(jnp.float32), axis=0, keepdims=True)
        keep = (rank < LEFT) & colv
        w = jnp.where(keep | (~colv), 0.0, a)
        rb = jnp.broadcast_to(rank, (NNP, NP))
        wb = jnp.broadcast_to(w, (NNP, NP))
        T = jnp.where((I2 >= 1) & (I2 <= LEFT) & (rb == (I2 - 1).astype(jnp.float32)),
                      1.0, 0.0)
        T = jnp.where((I2 == 0) & (J2 == 0), 1.0, T)
        T = jnp.where(I2 == LEFT + 1, wb, T)
        hs2[b * NNP:(b + 1) * NNP, :] = _dot_hi(T, hs[b * NP:(b + 1) * NP, :])


def _make_layer_args(i_off):
    def bs3(d1, d2):
        return pl.BlockSpec((1, d1, d2), lambda i: (i + i_off, 0, 0))

    def bs2(d1):
        return pl.BlockSpec((1, 1, d1), lambda i: (i + i_off, 0, 0))

    return [bs2(DIM), bs2(DIM), bs3(DIM, 3 * DIM), bs2(3 * DIM),
            bs3(DIM, DIM), bs2(DIM), bs2(DIM), bs2(DIM),
            bs3(DIM, 4 * DIM), bs2(4 * DIM), bs3(4 * DIM, DIM), bs2(DIM)]


def _const_bs(shape):
    nd = len(shape)
    return pl.BlockSpec(shape, lambda i, _n=nd: (0,) * _n)


def _unpack_layer(refs):
    (n1s, n1b, qw, qb, pw, pb, n2s, n2b, f1w, f1b, f2w, f2b) = refs
    return (n1s[0], n1b[0], qw[0], qb[0], pw[0], pb[0],
            n2s[0], n2b[0], f1w[0], f1b[0], f2w[0], f2b[0])


def _stage1(tok, patch_w, patch_b, clspos, posr, lw, NL, NP, N, LEFT, NNEW,
            NNP):
    M = B * NP
    CH = M // 8

    def body(tok_r, pw_r, pb_r, cp_r, pr_r,
             n1s, n1b, qw, qb, prw, prb, n2s, n2b, f1w, f1b, f2w, f2b,
             out_r, hs, qkvs, cls_scr, hs2):
        i = pl.program_id(0)

        @pl.when(i == 0)
        def _init():
            pwv = pw_r[...]
            pbv = pb_r[...]
            prv = pr_r[...]
            cpv = cp_r[...]
            for b in range(B):
                e = _dot(tok_r[b * (N - 1):(b + 1) * (N - 1), :], pwv) + pbv + prv
                hs[b * NP:b * NP + 1, :] = cpv
                hs[b * NP + 1:b * NP + N, :] = e
                if NP > N:
                    hs[b * NP + N:(b + 1) * NP, :] = jnp.zeros((NP - N, DIM),
                                                               jnp.float32)

        (n1sv, n1bv, qwv, qbv, prwv, prbv, n2sv, n2bv, f1wv, f1bv, f2wv,
         f2bv) = _unpack_layer((n1s, n1b, qw, qb, prw, prb, n2s, n2b, f1w,
                                f1b, f2w, f2b))
        _attn_and_proj(hs, qkvs, cls_scr, n1sv, n1bv, qwv, qbv, prwv,
                       prbv, NP, N, M, CH)

        @pl.when(i < NL - 1)
        def _mlp_mid():
            _mlp(hs, n2sv, n2bv, f1wv, f1bv, f2wv, f2bv, M, CH)

        @pl.when(i == NL - 1)
        def _prune_out():
            _prune(hs, hs2, cls_scr, NP, N, LEFT, NNP)
            M2 = B * NNP
            _mlp(hs2, n2sv, n2bv, f1wv, f1bv, f2wv, f2bv, M2, M2 // 8)
            for b in range(B):
                out_r[b, :, :] = hs2[b * NNP:b * NNP + NNEW, :]

    in_specs = [_const_bs(tok.shape), _const_bs(patch_w.shape),
                _const_bs(patch_b.shape), _const_bs(clspos.shape),
                _const_bs(posr.shape)] + _make_layer_args(0)
    return pl.pallas_call(
        body,
        grid=(NL,),
        in_specs=in_specs,
        out_specs=pl.BlockSpec((B, NNEW, DIM), lambda i: (0, 0, 0)),
        out_shape=jax.ShapeDtypeStruct((B, NNEW, DIM), jnp.float32),
        scratch_shapes=[
            pltpu.VMEM((M, DIM), jnp.float32),
            pltpu.VMEM((M, 3 * DIM), jnp.float32),
            pltpu.VMEM((B, NP), jnp.float32),
            pltpu.VMEM((B * NNP, DIM), jnp.float32),
        ],
        compiler_params=pltpu.CompilerParams(
            dimension_semantics=("arbitrary",)),
    )(tok, patch_w, patch_b, clspos, posr, *lw)


def _stage_mid(h_in, lw, OFF, NL, NP, N, LEFT, NNEW, NNP):
    M = B * NP
    CH = M // 8

    def body(h_r, n1s, n1b, qw, qb, prw, prb, n2s, n2b, f1w, f1b, f2w, f2b,
             out_r, hs, qkvs, cls_scr, hs2):
        i = pl.program_id(0)

        @pl.when(i == 0)
        def _init():
            for b in range(B):
                hs[b * NP:b * NP + N, :] = h_r[b, :, :]
                if NP > N:
                    hs[b * NP + N:(b + 1) * NP, :] = jnp.zeros((NP - N, DIM),
                                                               jnp.float32)

        vals = _unpack_layer((n1s, n1b, qw, qb, prw, prb, n2s, n2b, f1w, f1b,
                              f2w, f2b))
        (n1sv, n1bv, qwv, qbv, prwv, prbv, n2sv, n2bv, f1wv, f1bv, f2wv,
         f2bv) = vals
        _attn_and_proj(hs, qkvs, cls_scr, n1sv, n1bv, qwv, qbv, prwv,
                       prbv, NP, N, M, CH)

        @pl.when(i < NL - 1)
        def _mlp_mid():
            _mlp(hs, n2sv, n2bv, f1wv, f1bv, f2wv, f2bv, M, CH)

        @pl.when(i == NL - 1)
        def _prune_out():
            _prune(hs, hs2, cls_scr, NP, N, LEFT, NNP)
            M2 = B * NNP
            _mlp(hs2, n2sv, n2bv, f1wv, f1bv, f2wv, f2bv, M2, M2 // 8)
            for b in range(B):
                out_r[b, :, :] = hs2[b * NNP:b * NNP + NNEW, :]

    in_specs = [_const_bs(h_in.shape)] + _make_layer_args(OFF)
    return pl.pallas_call(
        body,
        grid=(NL,),
        in_specs=in_specs,
        out_specs=pl.BlockSpec((B, NNEW, DIM), lambda i: (0, 0, 0)),
        out_shape=jax.ShapeDtypeStruct((B, NNEW, DIM), jnp.float32),
        scratch_shapes=[
            pltpu.VMEM((M, DIM), jnp.float32),
            pltpu.VMEM((M, 3 * DIM), jnp.float32),
            pltpu.VMEM((B, NP), jnp.float32),
            pltpu.VMEM((B * NNP, DIM), jnp.float32),
        ],
        compiler_params=pltpu.CompilerParams(
            dimension_semantics=("arbitrary",)),
    )(h_in, *lw)


def _stage_final(h_in, lw, norm_s, norm_b, head_w, head_b, OFF, NL, NP, N,
                 NCLS):
    M = B * NP
    CH = M // 8

    def body(h_r, n1s, n1b, qw, qb, prw, prb, n2s, n2b, f1w, f1b, f2w, f2b,
             ns_r, nb_r, hw_r, hb_r, out_r, hs, qkvs, cls_scr, cm):
        i = pl.program_id(0)

        @pl.when(i == 0)
        def _init():
            for b in range(B):
                hs[b * NP:b * NP + N, :] = h_r[b, :, :]
                if NP > N:
                    hs[b * NP + N:(b + 1) * NP, :] = jnp.zeros((NP - N, DIM),
                                                               jnp.float32)

        (n1sv, n1bv, qwv, qbv, prwv, prbv, n2sv, n2bv, f1wv, f1bv, f2wv,
         f2bv) = _unpack_layer((n1s, n1b, qw, qb, prw, prb, n2s, n2b, f1w,
                                f1b, f2w, f2b))
        _attn_and_proj(hs, qkvs, cls_scr, n1sv, n1bv, qwv, qbv, prwv,
                       prbv, NP, N, M, CH)
        _mlp(hs, n2sv, n2bv, f1wv, f1bv, f2wv, f2bv, M, CH)

        @pl.when(i == NL - 1)
        def _head():
            for b in range(B):
                cm[b:b + 1, :] = hs[b * NP:b * NP + 1, :]
            z = _ln(cm[...], ns_r[...], nb_r[...])
            out_r[...] = _dot(z, hw_r[...]) + hb_r[...]

    in_specs = ([_const_bs(h_in.shape)] + _make_layer_args(OFF) +
                [_const_bs((1, DIM)), _const_bs((1, DIM)),
                 _const_bs(head_w.shape), _const_bs((1, NCLS))])
    return pl.pallas_call(
        body,
        grid=(NL,),
        in_specs=in_specs,
        out_specs=pl.BlockSpec((B, NCLS), lambda i: (0, 0)),
        out_shape=jax.ShapeDtypeStruct((B, NCLS), jnp.float32),
        scratch_shapes=[
            pltpu.VMEM((M, DIM), jnp.float32),
            pltpu.VMEM((M, 3 * DIM), jnp.float32),
            pltpu.VMEM((B, NP), jnp.float32),
            pltpu.VMEM((B, DIM), jnp.float32),
        ],
        compiler_params=pltpu.CompilerParams(
            dimension_semantics=("arbitrary",)),
    )(h_in, *lw, norm_s, norm_b, head_w, head_b)


def kernel(x, patch_w, patch_b, cls_token, pos_embed, norm1_s, norm1_b,
           qkv_w, qkv_b, proj_w, proj_b, norm2_s, norm2_b, fc1_w, fc1_b,
           fc2_w, fc2_b, norm_s, norm_b, head_w, head_b):
    gh = 14
    P = 16
    tok = x.reshape(B, 3, gh, P, gh, P).transpose(0, 2, 4, 1, 3, 5)
    tok = tok.reshape(B * gh * gh, 3 * P * P)
    clspos = (cls_token[0, 0] + pos_embed[0, 0]).reshape(1, DIM)
    posr = pos_embed[0, 1:]
    def r3(a):
        return a.reshape(12, 1, a.shape[-1])

    lw = (r3(norm1_s), r3(norm1_b), qkv_w, r3(qkv_b), proj_w,
          r3(proj_b), r3(norm2_s), r3(norm2_b), fc1_w, r3(fc1_b), fc2_w,
          r3(fc2_b))
    pb2 = patch_b.reshape(1, DIM)

    h = _stage1(tok, patch_w, pb2, clspos, posr, lw,
                NL=4, NP=200, N=197, LEFT=138, NNEW=140, NNP=144)
    h = _stage_mid(h, lw, OFF=4, NL=3, NP=144, N=140, LEFT=98, NNEW=100,
                   NNP=104)
    h = _stage_mid(h, lw, OFF=7, NL=3, NP=104, N=100, LEFT=70, NNEW=72,
                   NNP=72)
    out = _stage_final(h, lw, norm_s.reshape(1, DIM), norm_b.reshape(1, DIM),
                       head_w, head_b.reshape(1, 1000), OFF=10, NL=2, NP=72,
                       N=72, NCLS=1000)
    return out


# speedup vs baseline: 1.4589x; 1.4589x over previous
import jax
import jax.numpy as jnp
from jax.experimental import pallas as pl
from jax.experimental.pallas import tpu as pltpu

B = 8
DIM = 384
HEADS = 6
HD = 64
SCALE = HD ** -0.5
NEG = -1e30

_DN = (((1,), (0,)), ((), ()))
_DNT = (((1,), (1,)), ((), ()))


def _ln(x, s, b):
    m = jnp.mean(x, axis=-1, keepdims=True)
    d = x - m
    v = jnp.mean(d * d, axis=-1, keepdims=True)
    r = 1.0 / jnp.sqrt(v + 1e-5)
    return d * r * s + b


def _gelu(x):
    return 0.5 * x * (jax.lax.erf(x / 1.4142135623730951) + 1.0)


def _split(a):
    ah = a.astype(jnp.bfloat16)
    al = (a - ah.astype(jnp.float32)).astype(jnp.bfloat16)
    return ah, al


def _mm(x, y, dn=_DN):
    return jax.lax.dot_general(x, y, dn, preferred_element_type=jnp.float32)


def _dot3(a, bh, bl, dn=_DN):
    ah, al = _split(a)
    return _mm(ah, bh, dn) + (_mm(ah, bl, dn) + _mm(al, bh, dn))


def _dot3s(ah, al, bh, bl, dn=_DN):
    return _mm(ah, bh, dn) + (_mm(ah, bl, dn) + _mm(al, bh, dn))


def _attn_and_proj(hs, qkvs, ao, cls_scr, n1s, n1b, qw, qb, pw, pb,
                   NP, N, M, CH):
    qwh, qwl = _split(qw)
    for c0 in range(0, M, CH):
        x = hs[c0:c0 + CH, :]
        qkvs[c0:c0 + CH, :] = _dot3(_ln(x, n1s, n1b), qwh, qwl) + qb
    cls_scr[...] = jnp.zeros((B, NP), jnp.float32)
    if NP > N:
        kmask = jax.lax.broadcasted_iota(jnp.int32, (NP, NP), 1) >= N
    else:
        kmask = None
    for b in range(B):
        r0 = b * NP
        qkv_b = qkvs[r0:r0 + NP, :]
        qbh, qbl = _split(qkv_b)
        for h in range(HEADS):
            c_q = h * HD
            c_k = DIM + h * HD
            c_v = 2 * DIM + h * HD
            s = _dot3s(qbh[:, c_q:c_q + HD], qbl[:, c_q:c_q + HD],
                       qbh[:, c_k:c_k + HD], qbl[:, c_k:c_k + HD],
                       _DNT) * SCALE
            if kmask is not None:
                s = jnp.where(kmask, NEG, s)
            s = s - jnp.max(s, axis=1, keepdims=True)
            e = jnp.exp(s)
            inv = 1.0 / jnp.sum(e, axis=1, keepdims=True)
            cls_scr[b:b + 1, :] += e[0:1, :] * inv[0:1, :]
            ov = _dot3(e, qbh[:, c_v:c_v + HD], qbl[:, c_v:c_v + HD])
            ao[r0:r0 + NP, h * HD:(h + 1) * HD] = ov * inv
    pwh, pwl = _split(pw)
    for c0 in range(0, M, CH):
        hs[c0:c0 + CH, :] = (hs[c0:c0 + CH, :] +
                             _dot3(ao[c0:c0 + CH, :], pwh, pwl) + pb)


def _mlp(hs, n2s, n2b, f1h, f1l, f1b, f2h, f2l, f2b, M, CH):
    for c0 in range(0, M, CH):
        x = hs[c0:c0 + CH, :]
        z = _ln(x, n2s, n2b)
        z = _dot3(_gelu(_dot3(z, f1h, f1l) + f1b), f2h, f2l) + f2b
        hs[c0:c0 + CH, :] = x + z


def _prune(hs, hs2, cls_scr, NP, N, LEFT, NNP):
    ij1 = jax.lax.broadcasted_iota(jnp.int32, (1, NP), 1)
    colv = (ij1 >= 1) & (ij1 <= N - 1)
    ik = jax.lax.broadcasted_iota(jnp.int32, (NP, NP), 0)
    ij = jax.lax.broadcasted_iota(jnp.int32, (NP, NP), 1)
    I2 = jax.lax.broadcasted_iota(jnp.int32, (NNP, NP), 0)
    J2 = jax.lax.broadcasted_iota(jnp.int32, (NNP, NP), 1)
    for b in range(B):
        a = cls_scr[b:b + 1, :] * (1.0 / HEADS)
        av = jnp.where(colv, a, NEG)
        Mb = jnp.broadcast_to(av, (NP, NP))
        MT = Mb.T
        cnt = (MT > Mb) | ((MT == Mb) & (ik < ij))
        rank = jnp.sum(cnt.astype(jnp.float32), axis=0, keepdims=True)
        keep = (rank < LEFT) & colv
        w = jnp.where(keep | (~colv), 0.0, a)
        rb = jnp.broadcast_to(rank, (NNP, NP))
        wb = jnp.broadcast_to(w, (NNP, NP))
        T = jnp.where((I2 >= 1) & (I2 <= LEFT) &
                      (rb == (I2 - 1).astype(jnp.float32)), 1.0, 0.0)
        T = jnp.where((I2 == 0) & (J2 == 0), 1.0, T)
        T = jnp.where(I2 == LEFT + 1, wb, T)
        h_b = hs[b * NP:(b + 1) * NP, :]
        t1, tr = _split(T)
        t2, t3 = _split(tr.astype(jnp.float32))
        h1, r1 = _split(h_b)
        h2, h3 = _split(r1.astype(jnp.float32))
        out = (_mm(t1, h1) + (_mm(t1, h2) + _mm(t1, h3))
               + (_mm(t2, h1) + _mm(t2, h2)) + _mm(t3, h1))
        hs2[b * NNP:(b + 1) * NNP, :] = out


def _make_layer_args(i_off):
    def bs3(d1, d2):
        return pl.BlockSpec((1, d1, d2), lambda i: (i + i_off, 0, 0))

    def bs2(d1):
        return pl.BlockSpec((1, 1, d1), lambda i: (i + i_off, 0, 0))

    return [bs2(DIM), bs2(DIM), bs3(DIM, 3 * DIM), bs2(3 * DIM),
            bs3(DIM, DIM), bs2(DIM), bs2(DIM), bs2(DIM),
            bs3(DIM, 4 * DIM), bs2(4 * DIM), bs3(4 * DIM, DIM), bs2(DIM)]


def _const_bs(shape):
    nd = len(shape)
    return pl.BlockSpec(shape, lambda i, _n=nd: (0,) * _n)


def _unpack_layer(refs):
    (n1s, n1b, qw, qb, pw, pb, n2s, n2b, f1w, f1b, f2w, f2b) = refs
    return (n1s[0], n1b[0], qw[0], qb[0], pw[0], pb[0],
            n2s[0], n2b[0], f1w[0], f1b[0], f2w[0], f2b[0])


def _layer_and_out(i, NL, NP, N, LEFT, NNEW, NNP, M, CH, refs, out_r, hs,
                   qkvs, ao, cls_scr, hs2):
    (n1sv, n1bv, qwv, qbv, prwv, prbv, n2sv, n2bv, f1wv, f1bv, f2wv,
     f2bv) = _unpack_layer(refs)
    _attn_and_proj(hs, qkvs, ao, cls_scr, n1sv, n1bv, qwv, qbv, prwv,
                   prbv, NP, N, M, CH)
    f1h, f1l = _split(f1wv)
    f2h, f2l = _split(f2wv)

    @pl.when(i < NL - 1)
    def _mlp_mid():
        _mlp(hs, n2sv, n2bv, f1h, f1l, f1bv, f2h, f2l, f2bv, M, CH)

    @pl.when(i == NL - 1)
    def _prune_out():
        _prune(hs, hs2, cls_scr, NP, N, LEFT, NNP)
        M2 = B * NNP
        _mlp(hs2, n2sv, n2bv, f1h, f1l, f1bv, f2h, f2l, f2bv, M2, M2 // 4)
        for b in range(B):
            out_r[b, :, :] = hs2[b * NNP:b * NNP + NNEW, :]


def _stage1(tok, patch_w, patch_b, clspos, posr, lw, NL, NP, N, LEFT, NNEW,
            NNP):
    M = B * NP
    CH = M // 4

    def body(tok_r, pw_r, pb_r, cp_r, pr_r,
             n1s, n1b, qw, qb, prw, prb, n2s, n2b, f1w, f1b, f2w, f2b,
             out_r, hs, qkvs, ao, cls_scr, hs2):
        i = pl.program_id(0)

        @pl.when(i == 0)
        def _init():
            pwh, pwl = _split(pw_r[...])
            pbv = pb_r[...]
            prv = pr_r[...]
            cpv = cp_r[...]
            for b in range(B):
                e = _dot3(tok_r[b * (N - 1):(b + 1) * (N - 1), :],
                          pwh, pwl) + pbv + prv
                hs[b * NP:b * NP + 1, :] = cpv
                hs[b * NP + 1:b * NP + N, :] = e
                if NP > N:
                    hs[b * NP + N:(b + 1) * NP, :] = jnp.zeros(
                        (NP - N, DIM), jnp.float32)

        _layer_and_out(i, NL, NP, N, LEFT, NNEW, NNP, M, CH,
                       (n1s, n1b, qw, qb, prw, prb, n2s, n2b, f1w, f1b,
                        f2w, f2b), out_r, hs, qkvs, ao, cls_scr, hs2)

    in_specs = [_const_bs(tok.shape), _const_bs(patch_w.shape),
                _const_bs(patch_b.shape), _const_bs(clspos.shape),
                _const_bs(posr.shape)] + _make_layer_args(0)
    return pl.pallas_call(
        body,
        grid=(NL,),
        in_specs=in_specs,
        out_specs=pl.BlockSpec((B, NNEW, DIM), lambda i: (0, 0, 0)),
        out_shape=jax.ShapeDtypeStruct((B, NNEW, DIM), jnp.float32),
        scratch_shapes=[
            pltpu.VMEM((M, DIM), jnp.float32),
            pltpu.VMEM((M, 3 * DIM), jnp.float32),
            pltpu.VMEM((M, DIM), jnp.float32),
            pltpu.VMEM((B, NP), jnp.float32),
            pltpu.VMEM((B * NNP, DIM), jnp.float32),
        ],
        compiler_params=pltpu.CompilerParams(
            dimension_semantics=("arbitrary",)),
    )(tok, patch_w, patch_b, clspos, posr, *lw)


def _stage_mid(h_in, lw, OFF, NL, NP, N, LEFT, NNEW, NNP):
    M = B * NP
    CH = M // 4

    def body(h_r, n1s, n1b, qw, qb, prw, prb, n2s, n2b, f1w, f1b, f2w, f2b,
             out_r, hs, qkvs, ao, cls_scr, hs2):
        i = pl.program_id(0)

        @pl.when(i == 0)
        def _init():
            for b in range(B):
                hs[b * NP:b * NP + N, :] = h_r[b, :, :]
                if NP > N:
                    hs[b * NP + N:(b + 1) * NP, :] = jnp.zeros(
                        (NP - N, DIM), jnp.float32)

        _layer_and_out(i, NL, NP, N, LEFT, NNEW, NNP, M, CH,
                       (n1s, n1b, qw, qb, prw, prb, n2s, n2b, f1w, f1b,
                        f2w, f2b), out_r, hs, qkvs, ao, cls_scr, hs2)

    in_specs = [_const_bs(h_in.shape)] + _make_layer_args(OFF)
    return pl.pallas_call(
        body,
        grid=(NL,),
        in_specs=in_specs,
        out_specs=pl.BlockSpec((B, NNEW, DIM), lambda i: (0, 0, 0)),
        out_shape=jax.ShapeDtypeStruct((B, NNEW, DIM), jnp.float32),
        scratch_shapes=[
            pltpu.VMEM((M, DIM), jnp.float32),
            pltpu.VMEM((M, 3 * DIM), jnp.float32),
            pltpu.VMEM((M, DIM), jnp.float32),
            pltpu.VMEM((B, NP), jnp.float32),
            pltpu.VMEM((B * NNP, DIM), jnp.float32),
        ],
        compiler_params=pltpu.CompilerParams(
            dimension_semantics=("arbitrary",)),
    )(h_in, *lw)


def _stage_final(h_in, lw, norm_s, norm_b, head_w, head_b, OFF, NL, NP, N,
                 NCLS):
    M = B * NP
    CH = M // 4

    def body(h_r, n1s, n1b, qw, qb, prw, prb, n2s, n2b, f1w, f1b, f2w, f2b,
             ns_r, nb_r, hw_r, hb_r, out_r, hs, qkvs, ao, cls_scr, cm):
        i = pl.program_id(0)

        @pl.when(i == 0)
        def _init():
            for b in range(B):
                hs[b * NP:b * NP + N, :] = h_r[b, :, :]

        (n1sv, n1bv, qwv, qbv, prwv, prbv, n2sv, n2bv, f1wv, f1bv, f2wv,
         f2bv) = _unpack_layer((n1s, n1b, qw, qb, prw, prb, n2s, n2b, f1w,
                                f1b, f2w, f2b))
        _attn_and_proj(hs, qkvs, ao, cls_scr, n1sv, n1bv, qwv, qbv, prwv,
                       prbv, NP, N, M, CH)
        f1h, f1l = _split(f1wv)
        f2h, f2l = _split(f2wv)
        _mlp(hs, n2sv, n2bv, f1h, f1l, f1bv, f2h, f2l, f2bv, M, CH)

        @pl.when(i == NL - 1)
        def _head():
            for b in range(B):
                cm[b:b + 1, :] = hs[b * NP:b * NP + 1, :]
            z = _ln(cm[...], ns_r[...], nb_r[...])
            hwh, hwl = _split(hw_r[...])
            out_r[...] = _dot3(z, hwh, hwl) + hb_r[...]

    in_specs = ([_const_bs(h_in.shape)] + _make_layer_args(OFF) +
                [_const_bs((1, DIM)), _const_bs((1, DIM)),
                 _const_bs(head_w.shape), _const_bs((1, NCLS))])
    return pl.pallas_call(
        body,
        grid=(NL,),
        in_specs=in_specs,
        out_specs=pl.BlockSpec((B, NCLS), lambda i: (0, 0)),
        out_shape=jax.ShapeDtypeStruct((B, NCLS), jnp.float32),
        scratch_shapes=[
            pltpu.VMEM((M, DIM), jnp.float32),
            pltpu.VMEM((M, 3 * DIM), jnp.float32),
            pltpu.VMEM((M, DIM), jnp.float32),
            pltpu.VMEM((B, NP), jnp.float32),
            pltpu.VMEM((B, DIM), jnp.float32),
        ],
        compiler_params=pltpu.CompilerParams(
            dimension_semantics=("arbitrary",)),
    )(h_in, *lw, norm_s, norm_b, head_w, head_b)


def kernel(x, patch_w, patch_b, cls_token, pos_embed, norm1_s, norm1_b,
           qkv_w, qkv_b, proj_w, proj_b, norm2_s, norm2_b, fc1_w, fc1_b,
           fc2_w, fc2_b, norm_s, norm_b, head_w, head_b):
    gh = 14
    P = 16
    tok = x.reshape(B, 3, gh, P, gh, P).transpose(0, 2, 4, 1, 3, 5)
    tok = tok.reshape(B * gh * gh, 3 * P * P)
    clspos = (cls_token[0, 0] + pos_embed[0, 0]).reshape(1, DIM)
    posr = pos_embed[0, 1:]

    def r3(a):
        return a.reshape(12, 1, a.shape[-1])

    lw = (r3(norm1_s), r3(norm1_b), qkv_w, r3(qkv_b), proj_w,
          r3(proj_b), r3(norm2_s), r3(norm2_b), fc1_w, r3(fc1_b), fc2_w,
          r3(fc2_b))
    pb2 = patch_b.reshape(1, DIM)

    h = _stage1(tok, patch_w, pb2, clspos, posr, lw,
                NL=4, NP=200, N=197, LEFT=138, NNEW=140, NNP=144)
    h = _stage_mid(h, lw, OFF=4, NL=3, NP=144, N=140, LEFT=98, NNEW=100,
                   NNP=104)
    h = _stage_mid(h, lw, OFF=7, NL=3, NP=104, N=100, LEFT=70, NNEW=72,
                   NNP=72)
    out = _stage_final(h, lw, norm_s.reshape(1, DIM), norm_b.reshape(1, DIM),
                       head_w, head_b.reshape(1, 1000), OFF=10, NL=2, NP=72,
                       N=72, NCLS=1000)
    return out


# speedup vs baseline: 1.4862x; 1.0187x over previous
import jax
import jax.numpy as jnp
from jax.experimental import pallas as pl
from jax.experimental.pallas import tpu as pltpu

B = 8
DIM = 384
HEADS = 6
HD = 64
SCALE = HD ** -0.5
NEG = -1e30

_DN = (((1,), (0,)), ((), ()))
_DNT = (((1,), (1,)), ((), ()))


def _ln(x, s, b):
    m = jnp.mean(x, axis=-1, keepdims=True)
    d = x - m
    v = jnp.mean(d * d, axis=-1, keepdims=True)
    r = 1.0 / jnp.sqrt(v + 1e-5)
    return d * r * s + b


def _gelu(x):
    return 0.5 * x * (jax.lax.erf(x / 1.4142135623730951) + 1.0)


def _split(a):
    ah = a.astype(jnp.bfloat16)
    al = (a - ah.astype(jnp.float32)).astype(jnp.bfloat16)
    return ah, al


def _mm(x, y, dn=_DN):
    return jax.lax.dot_general(x, y, dn, preferred_element_type=jnp.float32)


def _dot3(a, bh, bl, dn=_DN):
    ah, al = _split(a)
    return _mm(ah, bh, dn) + (_mm(ah, bl, dn) + _mm(al, bh, dn))


def _dot3s(ah, al, bh, bl, dn=_DN):
    return _mm(ah, bh, dn) + (_mm(ah, bl, dn) + _mm(al, bh, dn))


def _attn_and_proj(hs, qkvs, ao, cls_scr, n1s, n1b, qw, qb, pw, pb,
                   NP, N, M, CH):
    qwh, qwl = _split(qw)
    for c0 in range(0, M, CH):
        x = hs[c0:c0 + CH, :]
        qkvs[c0:c0 + CH, :] = _dot3(_ln(x, n1s, n1b), qwh, qwl) + qb
    cls_scr[...] = jnp.zeros((B, NP), jnp.float32)
    if NP > N:
        kmask = jax.lax.broadcasted_iota(jnp.int32, (NP, NP), 1) >= N
    else:
        kmask = None
    for b in range(B):
        r0 = b * NP
        qkv_b = qkvs[r0:r0 + NP, :]
        qbh, qbl = _split(qkv_b)
        for h in range(HEADS):
            c_q = h * HD
            c_k = DIM + h * HD
            c_v = 2 * DIM + h * HD
            s = _dot3s(qbh[:, c_q:c_q + HD], qbl[:, c_q:c_q + HD],
                       qbh[:, c_k:c_k + HD], qbl[:, c_k:c_k + HD],
                       _DNT) * SCALE
            if kmask is not None:
                s = jnp.where(kmask, NEG, s)
            s = s - jnp.max(s, axis=1, keepdims=True)
            e = jnp.exp(s)
            inv = 1.0 / jnp.sum(e, axis=1, keepdims=True)
            cls_scr[b:b + 1, :] += e[0:1, :] * inv[0:1, :]
            ov = _dot3(e, qbh[:, c_v:c_v + HD], qbl[:, c_v:c_v + HD])
            ao[r0:r0 + NP, h * HD:(h + 1) * HD] = ov * inv
    pwh, pwl = _split(pw)
    for c0 in range(0, M, CH):
        hs[c0:c0 + CH, :] = (hs[c0:c0 + CH, :] +
                             _dot3(ao[c0:c0 + CH, :], pwh, pwl) + pb)


def _mlp(hs, n2s, n2b, f1h, f1l, f1b, f2h, f2l, f2b, M, CH):
    for c0 in range(0, M, CH):
        x = hs[c0:c0 + CH, :]
        z = _ln(x, n2s, n2b)
        z = _dot3(_gelu(_dot3(z, f1h, f1l) + f1b), f2h, f2l) + f2b
        hs[c0:c0 + CH, :] = x + z


def _prune(hs, hs2, cls_scr, NP, N, LEFT, NNP):
    ij1 = jax.lax.broadcasted_iota(jnp.int32, (1, NP), 1)
    colv = (ij1 >= 1) & (ij1 <= N - 1)
    ik = jax.lax.broadcasted_iota(jnp.int32, (NP, NP), 0)
    ij = jax.lax.broadcasted_iota(jnp.int32, (NP, NP), 1)
    I2 = jax.lax.broadcasted_iota(jnp.int32, (NNP, NP), 0)
    J2 = jax.lax.broadcasted_iota(jnp.int32, (NNP, NP), 1)
    for b in range(B):
        a = cls_scr[b:b + 1, :] * (1.0 / HEADS)
        av = jnp.where(colv, a, NEG)
        Mb = jnp.broadcast_to(av, (NP, NP))
        MT = Mb.T
        cnt = (MT > Mb) | ((MT == Mb) & (ik < ij))
        rank = jnp.sum(cnt.astype(jnp.float32), axis=0, keepdims=True)
        keep = (rank < LEFT) & colv
        w = jnp.where(keep | (~colv), 0.0, a)
        rb = jnp.broadcast_to(rank, (NNP, NP))
        wb = jnp.broadcast_to(w, (NNP, NP))
        T = jnp.where((I2 >= 1) & (I2 <= LEFT) &
                      (rb == (I2 - 1).astype(jnp.float32)), 1.0, 0.0)
        T = jnp.where((I2 == 0) & (J2 == 0), 1.0, T)
        T = jnp.where(I2 == LEFT + 1, wb, T)
        h_b = hs[b * NP:(b + 1) * NP, :]
        t1, tr = _split(T)
        t2, t3 = _split(tr.astype(jnp.float32))
        h1, r1 = _split(h_b)
        h2, h3 = _split(r1.astype(jnp.float32))
        out = (_mm(t1, h1) + (_mm(t1, h2) + _mm(t1, h3))
               + (_mm(t2, h1) + _mm(t2, h2)) + _mm(t3, h1))
        hs2[b * NNP:(b + 1) * NNP, :] = out


def _make_layer_args(i_off):
    def bs3(d1, d2):
        return pl.BlockSpec((1, d1, d2), lambda i: (i + i_off, 0, 0))

    def bs2(d1):
        return pl.BlockSpec((1, 1, d1), lambda i: (i + i_off, 0, 0))

    return [bs2(DIM), bs2(DIM), bs3(DIM, 3 * DIM), bs2(3 * DIM),
            bs3(DIM, DIM), bs2(DIM), bs2(DIM), bs2(DIM),
            bs3(DIM, 4 * DIM), bs2(4 * DIM), bs3(4 * DIM, DIM), bs2(DIM)]


def _const_bs(shape):
    nd = len(shape)
    return pl.BlockSpec(shape, lambda i, _n=nd: (0,) * _n)


def _unpack_layer(refs):
    (n1s, n1b, qw, qb, pw, pb, n2s, n2b, f1w, f1b, f2w, f2b) = refs
    return (n1s[0], n1b[0], qw[0], qb[0], pw[0], pb[0],
            n2s[0], n2b[0], f1w[0], f1b[0], f2w[0], f2b[0])


def _layer_and_out(i, NL, NP, N, LEFT, NNEW, NNP, M, CH, refs, out_r, hs,
                   qkvs, ao, cls_scr, hs2):
    (n1sv, n1bv, qwv, qbv, prwv, prbv, n2sv, n2bv, f1wv, f1bv, f2wv,
     f2bv) = _unpack_layer(refs)
    _attn_and_proj(hs, qkvs, ao, cls_scr, n1sv, n1bv, qwv, qbv, prwv,
                   prbv, NP, N, M, CH)
    f1h, f1l = _split(f1wv)
    f2h, f2l = _split(f2wv)

    @pl.when(i < NL - 1)
    def _mlp_mid():
        _mlp(hs, n2sv, n2bv, f1h, f1l, f1bv, f2h, f2l, f2bv, M, CH)

    @pl.when(i == NL - 1)
    def _prune_out():
        _prune(hs, hs2, cls_scr, NP, N, LEFT, NNP)
        M2 = B * NNP
        _mlp(hs2, n2sv, n2bv, f1h, f1l, f1bv, f2h, f2l, f2bv, M2, M2 // 4)
        for b in range(B):
            out_r[b, :, :] = hs2[b * NNP:b * NNP + NNEW, :]


def _stage1(tok, patch_w, patch_b, clspos, posr, lw, NL, NP, N, LEFT, NNEW,
            NNP):
    M = B * NP
    CH = M // 2

    def body(tok_r, pw_r, pb_r, cp_r, pr_r,
             n1s, n1b, qw, qb, prw, prb, n2s, n2b, f1w, f1b, f2w, f2b,
             out_r, hs, qkvs, ao, cls_scr, hs2):
        i = pl.program_id(0)

        @pl.when(i == 0)
        def _init():
            pwh, pwl = _split(pw_r[...])
            pbv = pb_r[...]
            prv = pr_r[...]
            cpv = cp_r[...]
            for b in range(B):
                e = _dot3(tok_r[b * (N - 1):(b + 1) * (N - 1), :],
                          pwh, pwl) + pbv + prv
                hs[b * NP:b * NP + 1, :] = cpv
                hs[b * NP + 1:b * NP + N, :] = e
                if NP > N:
                    hs[b * NP + N:(b + 1) * NP, :] = jnp.zeros(
                        (NP - N, DIM), jnp.float32)

        _layer_and_out(i, NL, NP, N, LEFT, NNEW, NNP, M, CH,
                       (n1s, n1b, qw, qb, prw, prb, n2s, n2b, f1w, f1b,
                        f2w, f2b), out_r, hs, qkvs, ao, cls_scr, hs2)

    in_specs = [_const_bs(tok.shape), _const_bs(patch_w.shape),
                _const_bs(patch_b.shape), _const_bs(clspos.shape),
                _const_bs(posr.shape)] + _make_layer_args(0)
    return pl.pallas_call(
        body,
        grid=(NL,),
        in_specs=in_specs,
        out_specs=pl.BlockSpec((B, NNEW, DIM), lambda i: (0, 0, 0)),
        out_shape=jax.ShapeDtypeStruct((B, NNEW, DIM), jnp.float32),
        scratch_shapes=[
            pltpu.VMEM((M, DIM), jnp.float32),
            pltpu.VMEM((M, 3 * DIM), jnp.float32),
            pltpu.VMEM((M, DIM), jnp.float32),
            pltpu.VMEM((B, NP), jnp.float32),
            pltpu.VMEM((B * NNP, DIM), jnp.float32),
        ],
        compiler_params=pltpu.CompilerParams(
            dimension_semantics=("arbitrary",)),
    )(tok, patch_w, patch_b, clspos, posr, *lw)


def _stage_mid(h_in, lw, OFF, NL, NP, N, LEFT, NNEW, NNP):
    M = B * NP
    CH = M // 2

    def body(h_r, n1s, n1b, qw, qb, prw, prb, n2s, n2b, f1w, f1b, f2w, f2b,
             out_r, hs, qkvs, ao, cls_scr, hs2):
        i = pl.program_id(0)

        @pl.when(i == 0)
        def _init():
            for b in range(B):
                hs[b * NP:b * NP + N, :] = h_r[b, :, :]
                if NP > N:
                    hs[b * NP + N:(b + 1) * NP, :] = jnp.zeros(
                        (NP - N, DIM), jnp.float32)

        _layer_and_out(i, NL, NP, N, LEFT, NNEW, NNP, M, CH,
                       (n1s, n1b, qw, qb, prw, prb, n2s, n2b, f1w, f1b,
                        f2w, f2b), out_r, hs, qkvs, ao, cls_scr, hs2)

    in_specs = [_const_bs(h_in.shape)] + _make_layer_args(OFF)
    return pl.pallas_call(
        body,
        grid=(NL,),
        in_specs=in_specs,
        out_specs=pl.BlockSpec((B, NNEW, DIM), lambda i: (0, 0, 0)),
        out_shape=jax.ShapeDtypeStruct((B, NNEW, DIM), jnp.float32),
        scratch_shapes=[
            pltpu.VMEM((M, DIM), jnp.float32),
            pltpu.VMEM((M, 3 * DIM), jnp.float32),
            pltpu.VMEM((M, DIM), jnp.float32),
            pltpu.VMEM((B, NP), jnp.float32),
            pltpu.VMEM((B * NNP, DIM), jnp.float32),
        ],
        compiler_params=pltpu.CompilerParams(
            dimension_semantics=("arbitrary",)),
    )(h_in, *lw)


def _stage_final(h_in, lw, norm_s, norm_b, head_w, head_b, OFF, NL, NP, N,
                 NCLS):
    M = B * NP
    CH = M // 2

    def body(h_r, n1s, n1b, qw, qb, prw, prb, n2s, n2b, f1w, f1b, f2w, f2b,
             ns_r, nb_r, hw_r, hb_r, out_r, hs, qkvs, ao, cls_scr, cm):
        i = pl.program_id(0)

        @pl.when(i == 0)
        def _init():
            for b in range(B):
                hs[b * NP:b * NP + N, :] = h_r[b, :, :]

        (n1sv, n1bv, qwv, qbv, prwv, prbv, n2sv, n2bv, f1wv, f1bv, f2wv,
         f2bv) = _unpack_layer((n1s, n1b, qw, qb, prw, prb, n2s, n2b, f1w,
                                f1b, f2w, f2b))
        _attn_and_proj(hs, qkvs, ao, cls_scr, n1sv, n1bv, qwv, qbv, prwv,
                       prbv, NP, N, M, CH)
        f1h, f1l = _split(f1wv)
        f2h, f2l = _split(f2wv)
        _mlp(hs, n2sv, n2bv, f1h, f1l, f1bv, f2h, f2l, f2bv, M, CH)

        @pl.when(i == NL - 1)
        def _head():
            for b in range(B):
                cm[b:b + 1, :] = hs[b * NP:b * NP + 1, :]
            z = _ln(cm[...], ns_r[...], nb_r[...])
            hwh, hwl = _split(hw_r[...])
            out_r[...] = _dot3(z, hwh, hwl) + hb_r[...]

    in_specs = ([_const_bs(h_in.shape)] + _make_layer_args(OFF) +
                [_const_bs((1, DIM)), _const_bs((1, DIM)),
                 _const_bs(head_w.shape), _const_bs((1, NCLS))])
    return pl.pallas_call(
        body,
        grid=(NL,),
        in_specs=in_specs,
        out_specs=pl.BlockSpec((B, NCLS), lambda i: (0, 0)),
        out_shape=jax.ShapeDtypeStruct((B, NCLS), jnp.float32),
        scratch_shapes=[
            pltpu.VMEM((M, DIM), jnp.float32),
            pltpu.VMEM((M, 3 * DIM), jnp.float32),
            pltpu.VMEM((M, DIM), jnp.float32),
            pltpu.VMEM((B, NP), jnp.float32),
            pltpu.VMEM((B, DIM), jnp.float32),
        ],
        compiler_params=pltpu.CompilerParams(
            dimension_semantics=("arbitrary",)),
    )(h_in, *lw, norm_s, norm_b, head_w, head_b)


def kernel(x, patch_w, patch_b, cls_token, pos_embed, norm1_s, norm1_b,
           qkv_w, qkv_b, proj_w, proj_b, norm2_s, norm2_b, fc1_w, fc1_b,
           fc2_w, fc2_b, norm_s, norm_b, head_w, head_b):
    gh = 14
    P = 16
    tok = x.reshape(B, 3, gh, P, gh, P).transpose(0, 2, 4, 1, 3, 5)
    tok = tok.reshape(B * gh * gh, 3 * P * P)
    clspos = (cls_token[0, 0] + pos_embed[0, 0]).reshape(1, DIM)
    posr = pos_embed[0, 1:]

    def r3(a):
        return a.reshape(12, 1, a.shape[-1])

    lw = (r3(norm1_s), r3(norm1_b), qkv_w, r3(qkv_b), proj_w,
          r3(proj_b), r3(norm2_s), r3(norm2_b), fc1_w, r3(fc1_b), fc2_w,
          r3(fc2_b))
    pb2 = patch_b.reshape(1, DIM)

    h = _stage1(tok, patch_w, pb2, clspos, posr, lw,
                NL=4, NP=200, N=197, LEFT=138, NNEW=140, NNP=144)
    h = _stage_mid(h, lw, OFF=4, NL=3, NP=144, N=140, LEFT=98, NNEW=100,
                   NNP=104)
    h = _stage_mid(h, lw, OFF=7, NL=3, NP=104, N=100, LEFT=70, NNEW=72,
                   NNP=72)
    out = _stage_final(h, lw, norm_s.reshape(1, DIM), norm_b.reshape(1, DIM),
                       head_w, head_b.reshape(1, 1000), OFF=10, NL=2, NP=72,
                       N=72, NCLS=1000)
    return out
